# Initial kernel scaffold; baseline (speedup 1.0000x reference)
import jax
import jax.numpy as jnp
from jax.experimental import pallas as pl


def kernel(x, adj, W1, b1, Wmu, bmu, Wlv, blv):
    raise NotImplementedError("write your pallas kernel here")


# speedup vs baseline: 1.3458x; 1.3458x over previous
import jax
import jax.numpy as jnp
from jax.experimental import pallas as pl
from jax.experimental.pallas import tpu as pltpu


def _pass1_kernel(x_ref, adj_ref, W1_ref, b1_ref, Wcat_ref, hm_ref, s1_ref):
    @pl.when(pl.program_id(0) == 0)
    def _():
        s1_ref[...] = jnp.dot(x_ref[...], W1_ref[...],
                              preferred_element_type=jnp.float32)

    h = jnp.dot(adj_ref[...], s1_ref[...], preferred_element_type=jnp.float32)
    h = jnp.maximum(h + b1_ref[...], 0.0)
    hm_ref[...] = jnp.dot(h, Wcat_ref[...], preferred_element_type=jnp.float32)


def _pass2_kernel(adj_ref, hm_ref, bcat_ref, out_ref):
    out_ref[...] = (
        jnp.dot(adj_ref[...], hm_ref[...], preferred_element_type=jnp.float32)
        + bcat_ref[...]
    )


def kernel(x, adj, W1, b1, Wmu, bmu, Wlv, blv):
    n, d = x.shape
    h_dim = W1.shape[1]
    e = Wmu.shape[1]

    Wcat = jnp.concatenate([Wmu, Wlv], axis=1)
    bcat = jnp.concatenate([bmu, blv])[None, :]
    b1r = b1[None, :]

    bm = 400
    nb = n // bm

    hm = pl.pallas_call(
        _pass1_kernel,
        grid=(nb,),
        in_specs=[
            pl.BlockSpec((n, d), lambda i: (0, 0)),
            pl.BlockSpec((bm, n), lambda i: (i, 0)),
            pl.BlockSpec((d, h_dim), lambda i: (0, 0)),
            pl.BlockSpec((1, h_dim), lambda i: (0, 0)),
            pl.BlockSpec((h_dim, 2 * e), lambda i: (0, 0)),
        ],
        out_specs=pl.BlockSpec((bm, 2 * e), lambda i: (i, 0)),
        out_shape=jax.ShapeDtypeStruct((n, 2 * e), jnp.float32),
        scratch_shapes=[pltpu.VMEM((n, h_dim), jnp.float32)],
    )(x, adj, W1, b1r, Wcat)

    out2 = pl.pallas_call(
        _pass2_kernel,
        grid=(nb,),
        in_specs=[
            pl.BlockSpec((bm, n), lambda i: (i, 0)),
            pl.BlockSpec((n, 2 * e), lambda i: (0, 0)),
            pl.BlockSpec((1, 2 * e), lambda i: (0, 0)),
        ],
        out_specs=pl.BlockSpec((bm, 2 * e), lambda i: (i, 0)),
        out_shape=jax.ShapeDtypeStruct((n, 2 * e), jnp.float32),
    )(adj, hm, bcat)

    mu = out2[:, :e]
    logvar = out2[:, e:]
    return (mu, mu, logvar)


# speedup vs baseline: 1.3522x; 1.0048x over previous
import jax
import jax.numpy as jnp
from jax.experimental import pallas as pl
from jax.experimental.pallas import tpu as pltpu


def _pass1_kernel(x_ref, adj_ref, W1_ref, b1_ref, Wcat_ref, hm_ref, s1_ref):
    @pl.when(pl.program_id(0) == 0)
    def _():
        s1_ref[...] = jnp.dot(x_ref[...], W1_ref[...],
                              preferred_element_type=jnp.float32
                              ).astype(jnp.bfloat16)

    h = jnp.dot(adj_ref[...].astype(jnp.bfloat16), s1_ref[...],
                preferred_element_type=jnp.float32)
    h = jnp.maximum(h + b1_ref[...], 0.0)
    hm_ref[...] = jnp.dot(h, Wcat_ref[...], preferred_element_type=jnp.float32
                          ).astype(jnp.bfloat16)


def _pass2_kernel(adj_ref, hm_ref, bcat_ref, out_ref):
    out_ref[...] = (
        jnp.dot(adj_ref[...].astype(jnp.bfloat16), hm_ref[...],
                preferred_element_type=jnp.float32)
        + bcat_ref[...]
    )


def kernel(x, adj, W1, b1, Wmu, bmu, Wlv, blv):
    n, d = x.shape
    h_dim = W1.shape[1]
    e = Wmu.shape[1]

    Wcat = jnp.concatenate([Wmu, Wlv], axis=1)
    bcat = jnp.concatenate([bmu, blv])[None, :]
    b1r = b1[None, :]

    bm = 400
    nb = n // bm

    hm = pl.pallas_call(
        _pass1_kernel,
        grid=(nb,),
        in_specs=[
            pl.BlockSpec((n, d), lambda i: (0, 0)),
            pl.BlockSpec((bm, n), lambda i: (i, 0)),
            pl.BlockSpec((d, h_dim), lambda i: (0, 0)),
            pl.BlockSpec((1, h_dim), lambda i: (0, 0)),
            pl.BlockSpec((h_dim, 2 * e), lambda i: (0, 0)),
        ],
        out_specs=pl.BlockSpec((bm, 2 * e), lambda i: (i, 0)),
        out_shape=jax.ShapeDtypeStruct((n, 2 * e), jnp.bfloat16),
        scratch_shapes=[pltpu.VMEM((n, h_dim), jnp.bfloat16)],
    )(x, adj, W1, b1r, Wcat)

    out2 = pl.pallas_call(
        _pass2_kernel,
        grid=(nb,),
        in_specs=[
            pl.BlockSpec((bm, n), lambda i: (i, 0)),
            pl.BlockSpec((n, 2 * e), lambda i: (0, 0)),
            pl.BlockSpec((1, 2 * e), lambda i: (0, 0)),
        ],
        out_specs=pl.BlockSpec((bm, 2 * e), lambda i: (i, 0)),
        out_shape=jax.ShapeDtypeStruct((n, 2 * e), jnp.float32),
    )(adj, hm, bcat)

    mu = out2[:, :e]
    logvar = out2[:, e:]
    return (mu, mu, logvar)


# speedup vs baseline: 1.3838x; 1.0233x over previous
import jax
import jax.numpy as jnp
from jax.experimental import pallas as pl
from jax.experimental.pallas import tpu as pltpu


def kernel(x, adj, W1, b1, Wmu, bmu, Wlv, blv):
    n, d = x.shape
    h_dim = W1.shape[1]
    e = Wmu.shape[1]

    Wcat = jnp.concatenate([Wmu, Wlv], axis=1)
    bcat = jnp.concatenate([bmu, blv])[None, :]
    b1r = b1[None, :]

    bm = 400
    nb = n // bm

    def fused_kernel(x_ref, adj_ref, W1_ref, b1_ref, Wcat_ref, bcat_ref,
                     out_ref, s1_ref, hm_ref):
        p = pl.program_id(0)
        i = pl.program_id(1)

        @pl.when((p == 0) & (i == 0))
        def _():
            s1_ref[...] = jnp.dot(
                x_ref[...], W1_ref[...],
                preferred_element_type=jnp.float32).astype(jnp.bfloat16)

        @pl.when(p == 0)
        def _():
            h = jnp.dot(adj_ref[...].astype(jnp.bfloat16), s1_ref[...],
                        preferred_element_type=jnp.float32)
            h = jnp.maximum(h + b1_ref[...], 0.0)
            hm_ref[pl.ds(i * bm, bm), :] = jnp.dot(
                h, Wcat_ref[...],
                preferred_element_type=jnp.float32).astype(jnp.bfloat16)

        @pl.when(p == 1)
        def _():
            out_ref[...] = jnp.dot(
                adj_ref[...].astype(jnp.bfloat16), hm_ref[...],
                preferred_element_type=jnp.float32) + bcat_ref[...]

    out2 = pl.pallas_call(
        fused_kernel,
        grid=(2, nb),
        in_specs=[
            pl.BlockSpec((n, d), lambda p, i: (0, 0)),
            pl.BlockSpec((bm, n), lambda p, i: (i, 0)),
            pl.BlockSpec((d, h_dim), lambda p, i: (0, 0)),
            pl.BlockSpec((1, h_dim), lambda p, i: (0, 0)),
            pl.BlockSpec((h_dim, 2 * e), lambda p, i: (0, 0)),
            pl.BlockSpec((1, 2 * e), lambda p, i: (0, 0)),
        ],
        out_specs=pl.BlockSpec((bm, 2 * e), lambda p, i: (p * i, 0)),
        out_shape=jax.ShapeDtypeStruct((n, 2 * e), jnp.float32),
        scratch_shapes=[
            pltpu.VMEM((n, h_dim), jnp.bfloat16),
            pltpu.VMEM((n, 2 * e), jnp.bfloat16),
        ],
    )(x, adj, W1, b1r, Wcat, bcat)

    mu = out2[:, :e]
    logvar = out2[:, e:]
    return (mu, mu, logvar)


# speedup vs baseline: 1.3850x; 1.0009x over previous
import jax
import jax.numpy as jnp
from jax.experimental import pallas as pl
from jax.experimental.pallas import tpu as pltpu


def kernel(x, adj, W1, b1, Wmu, bmu, Wlv, blv):
    n, d = x.shape
    h_dim = W1.shape[1]
    e = Wmu.shape[1]

    Wcat = jnp.concatenate([Wmu, Wlv], axis=1)
    bcat = jnp.concatenate([bmu, blv])[None, :]
    b1r = b1[None, :]

    bm = 400
    nb = n // bm

    def fused_kernel(x_ref, adj_ref, W1_ref, b1_ref, Wcat_ref, bcat_ref,
                     out_ref, s1_ref, hm_ref):
        p = pl.program_id(0)
        i = pl.program_id(1)

        @pl.when((p == 0) & (i == 0))
        def _():
            s1_ref[...] = jnp.dot(
                x_ref[...], W1_ref[...],
                preferred_element_type=jnp.float32).astype(jnp.bfloat16)

        @pl.when(p == 0)
        def _():
            h = jnp.dot(adj_ref[...], s1_ref[...].astype(jnp.float32),
                        preferred_element_type=jnp.float32)
            h = jnp.maximum(h + b1_ref[...], 0.0)
            hm_ref[pl.ds(i * bm, bm), :] = jnp.dot(
                h, Wcat_ref[...],
                preferred_element_type=jnp.float32).astype(jnp.bfloat16)

        @pl.when(p == 1)
        def _():
            out_ref[...] = jnp.dot(
                adj_ref[...], hm_ref[...].astype(jnp.float32),
                preferred_element_type=jnp.float32) + bcat_ref[...]

    out2 = pl.pallas_call(
        fused_kernel,
        grid=(2, nb),
        in_specs=[
            pl.BlockSpec((n, d), lambda p, i: (0, 0)),
            pl.BlockSpec((bm, n), lambda p, i: (i, 0)),
            pl.BlockSpec((d, h_dim), lambda p, i: (0, 0)),
            pl.BlockSpec((1, h_dim), lambda p, i: (0, 0)),
            pl.BlockSpec((h_dim, 2 * e), lambda p, i: (0, 0)),
            pl.BlockSpec((1, 2 * e), lambda p, i: (0, 0)),
        ],
        out_specs=pl.BlockSpec((bm, 2 * e), lambda p, i: (p * i, 0)),
        out_shape=jax.ShapeDtypeStruct((n, 2 * e), jnp.float32),
        scratch_shapes=[
            pltpu.VMEM((n, h_dim), jnp.bfloat16),
            pltpu.VMEM((n, 2 * e), jnp.bfloat16),
        ],
    )(x, adj, W1, b1r, Wcat, bcat)

    mu = out2[:, :e]
    logvar = out2[:, e:]
    return (mu, mu, logvar)
